# baseline (speedup 1.0000x reference)
import jax
import jax.numpy as jnp
from jax import lax
from jax.experimental import pallas as pl
from jax.experimental.pallas import tpu as pltpu
from jax.experimental.pallas import tpu_sc as plsc

NUM_WORKERS = 32
CHUNK = 64
N_CHUNKS = 40
N_PAIRS = N_CHUNKS // 2


def _make_sc_gather(n_rows: int, d: int):
    per_w = n_rows // NUM_WORKERS
    assert per_w == N_CHUNKS * CHUNK

    mesh = plsc.VectorSubcoreMesh(core_axis_name="c", subcore_axis_name="s")

    @pl.kernel(
        mesh=mesh,
        compiler_params=pltpu.CompilerParams(use_tc_tiling_on_sc=False),
        out_type=jax.ShapeDtypeStruct((n_rows, d), jnp.float32),
        scratch_types=[
            pltpu.VMEM((N_CHUNKS, CHUNK), jnp.int32),
            pltpu.VMEM((CHUNK, d), jnp.float32),
            pltpu.VMEM((CHUNK, d), jnp.float32),
            pltpu.SemaphoreType.DMA,
            pltpu.SemaphoreType.DMA,
            pltpu.SemaphoreType.DMA,
            pltpu.SemaphoreType.DMA,
        ],
    )
    def sc_gather(table_hbm, idx_hbm, out_hbm, idx_v, rows_a, rows_b,
                  sem_ga, sem_gb, sem_wa, sem_wb):
        wid = lax.axis_index("s") * 2 + lax.axis_index("c")
        base = wid * per_w
        pltpu.sync_copy(idx_hbm.at[wid], idx_v)

        def gather(j, buf, sem):
            return pltpu.make_async_copy(table_hbm.at[idx_v.at[j]], buf, sem)

        def write(j, buf, sem):
            return pltpu.make_async_copy(
                buf, out_hbm.at[pl.ds(base + j * CHUNK, CHUNK)], sem)

        gather(0, rows_a, sem_ga).start()

        def body(k, carry):
            j0 = 2 * k
            gather(j0, rows_a, sem_ga).wait()
            write(j0, rows_a, sem_wa).start()

            @pl.when(k > 0)
            def _():
                write(j0 - 1, rows_b, sem_wb).wait()

            gather(j0 + 1, rows_b, sem_gb).start()
            gather(j0 + 1, rows_b, sem_gb).wait()
            write(j0 + 1, rows_b, sem_wb).start()
            write(j0, rows_a, sem_wa).wait()

            @pl.when(k < N_PAIRS - 1)
            def _():
                gather(j0 + 2, rows_a, sem_ga).start()
            return carry

        lax.fori_loop(0, N_PAIRS, body, 0)
        write(N_CHUNKS - 1, rows_b, sem_wb).wait()

    return sc_gather


def kernel(x, embedding):
    b, t = x.shape
    n = b * t
    d = embedding.shape[1]
    idx = x.reshape(-1).astype(jnp.int32).reshape(NUM_WORKERS, N_CHUNKS, CHUNK)
    out = _make_sc_gather(n, d)(embedding, idx)
    return out.reshape(b, t, d)


# speedup vs baseline: 1.0015x; 1.0015x over previous
import jax
import jax.numpy as jnp
from jax import lax
from jax.experimental import pallas as pl
from jax.experimental.pallas import tpu as pltpu
from jax.experimental.pallas import tpu_sc as plsc

NUM_WORKERS = 32
CHUNK = 40
N_CHUNKS = 64
N_PAIRS = N_CHUNKS // 2


def _make_sc_gather(b: int, t: int, d: int):
    n_rows = b * t
    per_w = n_rows // NUM_WORKERS
    assert per_w == N_CHUNKS * CHUNK
    b_per_chunk = CHUNK // t

    mesh = plsc.VectorSubcoreMesh(core_axis_name="c", subcore_axis_name="s")

    @pl.kernel(
        mesh=mesh,
        compiler_params=pltpu.CompilerParams(use_tc_tiling_on_sc=False),
        out_type=jax.ShapeDtypeStruct((b, t, d), jnp.float32),
        scratch_types=[
            pltpu.VMEM((N_CHUNKS, CHUNK), jnp.int32),
            pltpu.VMEM((CHUNK, d), jnp.float32),
            pltpu.VMEM((CHUNK, d), jnp.float32),
            pltpu.SemaphoreType.DMA,
            pltpu.SemaphoreType.DMA,
            pltpu.SemaphoreType.DMA,
            pltpu.SemaphoreType.DMA,
        ],
    )
    def sc_gather(table_hbm, idx_hbm, out_hbm, idx_v, rows_a, rows_b,
                  sem_ga, sem_gb, sem_wa, sem_wb):
        wid = lax.axis_index("s") * 2 + lax.axis_index("c")
        b_base = wid * (per_w // t)
        pltpu.sync_copy(idx_hbm.at[wid], idx_v)

        def gather(j, buf, sem):
            return pltpu.make_async_copy(table_hbm.at[idx_v.at[j]], buf, sem)

        def write_piece(j, buf, sem, i):
            bb = b_base + j * b_per_chunk + i
            return pltpu.make_async_copy(
                buf.at[pl.ds(i * t, t)], out_hbm.at[bb], sem)

        def write_start(j, buf, sem):
            for i in range(b_per_chunk):
                write_piece(j, buf, sem, i).start()

        def write_wait(j, buf, sem):
            for i in range(b_per_chunk):
                write_piece(j, buf, sem, i).wait()

        gather(0, rows_a, sem_ga).start()

        def body(k, carry):
            j0 = 2 * k
            gather(j0, rows_a, sem_ga).wait()
            write_start(j0, rows_a, sem_wa)

            @pl.when(k > 0)
            def _():
                write_wait(j0 - 1, rows_b, sem_wb)

            gather(j0 + 1, rows_b, sem_gb).start()
            gather(j0 + 1, rows_b, sem_gb).wait()
            write_start(j0 + 1, rows_b, sem_wb)
            write_wait(j0, rows_a, sem_wa)

            @pl.when(k < N_PAIRS - 1)
            def _():
                gather(j0 + 2, rows_a, sem_ga).start()
            return carry

        lax.fori_loop(0, N_PAIRS, body, 0)
        write_wait(N_CHUNKS - 1, rows_b, sem_wb)

    return sc_gather


def kernel(x, embedding):
    b, t = x.shape
    d = embedding.shape[1]
    idx = x.reshape(-1).astype(jnp.int32).reshape(NUM_WORKERS, N_CHUNKS, CHUNK)
    return _make_sc_gather(b, t, d)(embedding, idx)


# speedup vs baseline: 1.2843x; 1.2823x over previous
import jax
import jax.numpy as jnp
from jax import lax
from jax.experimental import pallas as pl
from jax.experimental.pallas import tpu as pltpu
from jax.experimental.pallas import tpu_sc as plsc

NUM_WORKERS = 32
BPW = 128
NCB = 8
D_PAD = NCB * 128


def _make_sc_gather(b, t, d):
    d_tail = d - (NCB - 1) * 128
    mesh = plsc.VectorSubcoreMesh(core_axis_name="c", subcore_axis_name="s")

    @pl.kernel(
        mesh=mesh,
        out_type=jax.ShapeDtypeStruct((b, t, d), jnp.float32),
        scratch_types=[
            pltpu.VMEM((t, NCB, BPW), jnp.int32),
            pltpu.VMEM((BPW, 128), jnp.float32),
            pltpu.VMEM((BPW, 128), jnp.float32),
            pltpu.VMEM((BPW, d_tail), jnp.float32),
            pltpu.VMEM((BPW, d_tail), jnp.float32),
            pltpu.SemaphoreType.DMA,
            pltpu.SemaphoreType.DMA,
            pltpu.SemaphoreType.DMA,
            pltpu.SemaphoreType.DMA,
            pltpu.SemaphoreType.DMA,
            pltpu.SemaphoreType.DMA,
        ],
    )
    def sc_gather(t8_hbm, idx_hbm, out_hbm, idx_v, pa, pb, buf7a, buf7b,
                  sem_ga, sem_gb, sem_wa, sem_wb, sem_7a, sem_7b):
        wid = lax.axis_index("s") * 2 + lax.axis_index("c")
        bb = wid * BPW
        pltpu.sync_copy(idx_hbm.at[wid], idx_v)

        def gather(tt, c, buf, sem):
            return pltpu.make_async_copy(t8_hbm.at[idx_v.at[tt, c]], buf, sem)

        def write(tt, c, buf, sem):
            return pltpu.make_async_copy(
                buf, out_hbm.at[pl.ds(bb, BPW), tt, pl.ds(c * 128, 128)], sem)

        def write7(tt, bf, sem):
            return pltpu.make_async_copy(
                bf, out_hbm.at[pl.ds(bb, BPW), tt, pl.ds((NCB - 1) * 128,
                                                         d_tail)], sem)

        def tail(tt, bf, sem):
            @pl.when(tt >= 2)
            def _():
                write7(tt - 2, bf, sem).wait()

            def row_copy(r, carry):
                for kk in range(d_tail // 16):
                    bf[r, pl.ds(kk * 16, 16)] = pb[r, pl.ds(kk * 16, 16)]
                bf[r, pl.ds(d_tail - 16, 16)] = pb[r, pl.ds(d_tail - 16, 16)]
                return carry
            lax.fori_loop(0, BPW, row_copy, 0)
            write7(tt, bf, sem).start()

        gather(0, 0, pa, sem_ga).start()

        def body(tt, carry):
            for p in range(4):
                ca, cb = 2 * p, 2 * p + 1
                gather(tt, ca, pa, sem_ga).wait()
                write(tt, ca, pa, sem_wa).start()
                if p > 0:
                    write(tt, cb - 2, pb, sem_wb).wait()
                gather(tt, cb, pb, sem_gb).start()
                gather(tt, cb, pb, sem_gb).wait()
                if p < 3:
                    write(tt, cb, pb, sem_wb).start()
                else:
                    @pl.when(tt % 2 == 0)
                    def _():
                        tail(tt, buf7a, sem_7a)

                    @pl.when(tt % 2 == 1)
                    def _():
                        tail(tt, buf7b, sem_7b)
                write(tt, ca, pa, sem_wa).wait()
                if p < 3:
                    gather(tt, ca + 2, pa, sem_ga).start()
                else:
                    @pl.when(tt < t - 1)
                    def _():
                        gather(tt + 1, 0, pa, sem_ga).start()
            return carry

        lax.fori_loop(0, t, body, 0)
        write7(t - 2, buf7a, sem_7a).wait()
        write7(t - 1, buf7b, sem_7b).wait()

    return sc_gather


def kernel(x, embedding):
    b, t = x.shape
    v, d = embedding.shape
    t8 = jnp.pad(embedding, ((0, 0), (0, D_PAD - d))).reshape(v * NCB, 128)
    xr = x.astype(jnp.int32).reshape(NUM_WORKERS, BPW, t).transpose(0, 2, 1)
    gidx = (xr[:, :, None, :] * NCB
            + jnp.arange(NCB, dtype=jnp.int32)[None, None, :, None])
    return _make_sc_gather(b, t, d)(t8, gidx)
